# baseline (speedup 1.0000x reference)
import jax
import jax.numpy as jnp
from jax.experimental import pallas as pl
from jax.experimental.pallas import tpu as pltpu

_N = 16384
_NBINS = 50
_RB = 48
_NRB = 21
_NCHUNK = 8
_NC = _N // _NCHUNK
_SYNC_COST = 10.0
_TARGET = 0.1


def _body(x_ref, out_ref, stage_ref, acc_ref):
    j = pl.program_id(0)
    r = pl.program_id(1)

    @pl.when((j == 0) & (r == 0))
    def _():
        acc_ref[...] = jnp.zeros_like(acc_ref)

    row0 = pl.multiple_of(_RB * r, _RB)
    stage_ref[pl.ds(row0, _RB), :] = x_ref[0]

    @pl.when(r == _NRB - 1)
    def _():
        y = stage_ref[pl.ds(2, _NBINS * 20), :]
        binned = y.reshape(_NBINS, 20, _NC)
        sums = jnp.sum(binned, axis=1)
        acc_ref[...] = acc_ref[...] + (sums != 0.0).astype(jnp.float32)

    @pl.when((j == _NCHUNK - 1) & (r == _NRB - 1))
    def _():
        counts = jnp.sum(acc_ref[...], axis=1, keepdims=True)
        m = jnp.max(counts)
        frac = m / jnp.float32(_N)
        d = frac - jnp.float32(_TARGET)
        out_ref[0, 0] = jnp.float32(_SYNC_COST) * d * d


def kernel(spikes):
    out = pl.pallas_call(
        _body,
        grid=(_NCHUNK, _NRB),
        in_specs=[
            pl.BlockSpec((1, _RB, _NC), lambda j, r: (0, r + 1, j))
        ],
        out_specs=pl.BlockSpec(memory_space=pltpu.SMEM),
        out_shape=jax.ShapeDtypeStruct((1, 1), jnp.float32),
        scratch_shapes=[
            pltpu.VMEM((_NRB * _RB, _NC), jnp.float32),
            pltpu.VMEM((_NBINS, _NC), jnp.float32),
        ],
    )(spikes)
    return out[0, 0]


# speedup vs baseline: 1.2008x; 1.2008x over previous
import jax
import jax.numpy as jnp
from jax.experimental import pallas as pl
from jax.experimental.pallas import tpu as pltpu

_N = 16384
_NBINS = 50
_G = 21
_NCHUNK = 8
_NC = _N // _NCHUNK
_SYNC_COST = 10.0
_TARGET = 0.1


def _body(x_ref, out_ref, acc_ref):
    j = pl.program_id(0)

    @pl.when(j == 0)
    def _():
        acc_ref[...] = jnp.zeros_like(acc_ref)

    x = x_ref[0]
    y = x[1:].reshape(_NBINS, 20, _NC)
    sums = jnp.sum(y, axis=1)
    acc_ref[...] = acc_ref[...] + (sums != 0.0).astype(jnp.float32)

    @pl.when(j == _NCHUNK - 1)
    def _():
        counts = jnp.sum(acc_ref[...], axis=1, keepdims=True)
        m = jnp.max(counts)
        frac = m / jnp.float32(_N)
        d = frac - jnp.float32(_TARGET)
        out_ref[0, 0] = jnp.float32(_SYNC_COST) * d * d


def kernel(spikes):
    x4 = spikes.reshape(4, 22, 50, _N)
    out = pl.pallas_call(
        _body,
        grid=(_NCHUNK,),
        in_specs=[
            pl.BlockSpec((1, _G, 50, _NC), lambda j: (0, 0, 0, j))
        ],
        out_specs=pl.BlockSpec(memory_space=pltpu.SMEM),
        out_shape=jax.ShapeDtypeStruct((1, 1), jnp.float32),
        scratch_shapes=[
            pltpu.VMEM((_NBINS, _NC), jnp.float32),
        ],
    )(x4)
    return out[0, 0]


# speedup vs baseline: 1.3286x; 1.1065x over previous
import jax
import jax.numpy as jnp
from jax.experimental import pallas as pl
from jax.experimental.pallas import tpu as pltpu

_N = 16384
_NBINS = 50
_ROWS = 1056
_NCHUNK = 8
_NC = _N // _NCHUNK
_SYNC_COST = 10.0
_TARGET = 0.1


def _body(x_ref, out_ref, acc_ref):
    j = pl.program_id(0)

    @pl.when(j == 0)
    def _():
        acc_ref[...] = jnp.zeros_like(acc_ref)

    x = x_ref[0]
    binned = x[50:50 + _NBINS * 20, :].reshape(_NBINS, 20, _NC)
    sums = jnp.sum(binned, axis=1)
    acc_ref[...] = acc_ref[...] + (sums != 0.0).astype(jnp.float32)

    @pl.when(j == _NCHUNK - 1)
    def _():
        counts = jnp.sum(acc_ref[...], axis=1, keepdims=True)
        m = jnp.max(counts)
        frac = m / jnp.float32(_N)
        d = frac - jnp.float32(_TARGET)
        out_ref[0, 0] = jnp.float32(_SYNC_COST) * d * d


def kernel(spikes):
    out = pl.pallas_call(
        _body,
        grid=(_NCHUNK,),
        in_specs=[
            pl.BlockSpec((1, _ROWS, _NC), lambda j: (0, 0, j))
        ],
        out_specs=pl.BlockSpec(memory_space=pltpu.SMEM),
        out_shape=jax.ShapeDtypeStruct((1, 1), jnp.float32),
        scratch_shapes=[
            pltpu.VMEM((_NBINS, _NC), jnp.float32),
        ],
    )(spikes)
    return out[0, 0]
